# Initial kernel scaffold; baseline (speedup 1.0000x reference)
import jax
import jax.numpy as jnp
from jax.experimental import pallas as pl


def kernel(candidate_edges, loglog_u, sampled_edges, edges_logits):
    raise NotImplementedError("write your pallas kernel here")


# speedup vs baseline: 18.3143x; 18.3143x over previous
import functools

import jax
import jax.numpy as jnp
from jax import lax
from jax.experimental import pallas as pl
from jax.experimental.pallas import tpu as pltpu
from jax.experimental.pallas import tpu_sc as plsc

NFE = 6400000
NCAND = 1000000
NSAMP = 200000
NSEG = 1024

NC, NS, L = 2, 16, 16
NW = NC * NS

UC = NCAND // 64
UW = UC // NW
UC_EXTRA = UC - UW * NW
US = NSAMP // 64
SW = US // NW
US_EXTRA = US - SW * NW

_mesh = plsc.VectorSubcoreMesh(
    core_axis_name="c", subcore_axis_name="s",
    num_cores=NC, num_subcores=NS)

_NEG_BIG = -3.0e38

_params = pltpu.CompilerParams(needs_layout_passes=False, use_tc_tiling_on_sc=False)


def _wid():
    return lax.axis_index("s") * NC + lax.axis_index("c")


def _global_max(pm_all):
    def mrow(r, mx):
        return jnp.maximum(mx, pm_all[r, :])
    mx16 = lax.fori_loop(0, NW, mrow, jnp.full((L,), _NEG_BIG, jnp.float32))
    return jnp.max(mx16)


@functools.partial(
    pl.kernel,
    out_type=(
        jax.ShapeDtypeStruct((UC, 64), jnp.float32),
        jax.ShapeDtypeStruct((UC, 64), jnp.int32),
        jax.ShapeDtypeStruct((NW, L), jnp.float32),
    ),
    mesh=_mesh,
    compiler_params=_params,
    scratch_types=(
        pltpu.VMEM((2560,), jnp.int32),
        pltpu.VMEM((UW, 64), jnp.int32),
        pltpu.VMEM((UW, 64), jnp.int32),
        pltpu.VMEM((UW, 64), jnp.float32),
        pltpu.VMEM((UW, 64), jnp.float32),
        pltpu.VMEM((320,), jnp.int32),
        pltpu.VMEM((1, 64), jnp.int32),
        pltpu.VMEM((1, 64), jnp.int32),
        pltpu.VMEM((1, 64), jnp.float32),
        pltpu.VMEM((1, 64), jnp.float32),
        pltpu.VMEM((L,), jnp.float32),
        pltpu.VMEM((L,), jnp.float32),
        pltpu.SemaphoreType.DMA,
    ),
)
def _k1(cand_hbm, u_hbm, logits_hbm, z_hbm, eg_hbm, pmax_hbm,
        rows_v, ei_v, eg_v, lg_v, u_v,
        ex_rows, ex_ei, ex_eg, ex_lg, ex_u, pm_v, max_ref, sem):
    wid = _wid()
    u0 = wid * UW
    iota = lax.iota(jnp.int32, L)
    max_ref[:] = jnp.full((L,), _NEG_BIG, jnp.float32)

    def batch_body(b, _):
        pltpu.sync_copy(cand_hbm.at[pl.ds((u0 + b * 8) * 320, 2560)], rows_v)
        for un in range(8):
            for j in range(4):
                base16 = (un * 64 + j * L + iota) * 5
                g_eg = plsc.load_gather(rows_v, [base16])
                g_ei = plsc.load_gather(rows_v, [base16 + 1])
                eg_v[b * 8 + un, pl.ds(j * L, L)] = g_eg
                ei_v[b * 8 + un, pl.ds(j * L, L)] = g_ei
        return 0
    lax.fori_loop(0, UW // 8, batch_body, 0)

    def fire(r, _):
        pltpu.async_copy(logits_hbm.at[ei_v.at[r]], lg_v.at[r], sem)
        return 0
    lax.fori_loop(0, UW, fire, 0)
    pltpu.sync_copy(u_hbm.at[pl.ds(u0, UW)], u_v)

    @pl.when(wid < UC_EXTRA)
    def _extra():
        eu = UW * NW + wid
        pltpu.sync_copy(cand_hbm.at[pl.ds(eu * 320, 320)], ex_rows)
        for j in range(4):
            base16 = (j * L + iota) * 5
            ex_eg[0, pl.ds(j * L, L)] = plsc.load_gather(ex_rows, [base16])
            ex_ei[0, pl.ds(j * L, L)] = plsc.load_gather(ex_rows, [base16 + 1])
        pltpu.sync_copy(u_hbm.at[pl.ds(eu, 1)], ex_u)
        pltpu.sync_copy(logits_hbm.at[ex_ei.at[0]], ex_lg.at[0])
        for j in range(4):
            zj = ex_lg[0, pl.ds(j * L, L)] + ex_u[0, pl.ds(j * L, L)]
            ex_lg[0, pl.ds(j * L, L)] = zj
            max_ref[:] = jnp.maximum(max_ref[:], zj)
        pltpu.sync_copy(ex_lg, z_hbm.at[pl.ds(eu, 1)])
        pltpu.sync_copy(ex_eg, eg_hbm.at[pl.ds(eu, 1)])

    def drain(r, _):
        pltpu.make_async_copy(logits_hbm.at[ei_v.at[r]], lg_v.at[r], sem).wait()
        return 0
    lax.fori_loop(0, UW, drain, 0)

    def zrow(r, mx):
        for cj in range(4):
            zj = lg_v[r, pl.ds(cj * L, L)] + u_v[r, pl.ds(cj * L, L)]
            lg_v[r, pl.ds(cj * L, L)] = zj
            mx = jnp.maximum(mx, zj)
        return mx
    mx = lax.fori_loop(0, UW, zrow, max_ref[:])

    pm_v[:] = mx
    pltpu.sync_copy(pm_v, pmax_hbm.at[wid])
    pltpu.sync_copy(lg_v, z_hbm.at[pl.ds(u0, UW)])
    pltpu.sync_copy(eg_v, eg_hbm.at[pl.ds(u0, UW)])


@functools.partial(
    pl.kernel,
    out_type=jax.ShapeDtypeStruct((NC, NSEG), jnp.float32),
    mesh=_mesh,
    compiler_params=_params,
    scratch_types=(
        pltpu.VMEM((NW, L), jnp.float32),
        pltpu.VMEM((UW, 64), jnp.float32),
        pltpu.VMEM((UW, 64), jnp.int32),
        pltpu.VMEM((1, 64), jnp.float32),
        pltpu.VMEM((1, 64), jnp.int32),
        pltpu.VMEM((NSEG,), jnp.float32),
        pltpu.VMEM_SHARED((NSEG,), jnp.float32),
        pltpu.SemaphoreType.DMA,
    ),
)
def _k2(z_hbm, eg_hbm, pmax_hbm, sums_hbm,
        pm_all, z_v, eg_v, ex_z, ex_eg, zeros_v, shared, sem):
    wid = _wid()
    u0 = wid * UW
    pltpu.sync_copy(pmax_hbm, pm_all)
    m = _global_max(pm_all)

    pltpu.sync_copy(z_hbm.at[pl.ds(u0, UW)], z_v)
    pltpu.sync_copy(eg_hbm.at[pl.ds(u0, UW)], eg_v)

    for t in range(NSEG // L):
        zeros_v[pl.ds(t * L, L)] = jnp.zeros((L,), jnp.float32)

    @pl.when(lax.axis_index("s") == 0)
    def _init():
        pltpu.sync_copy(zeros_v, shared)
    plsc.subcore_barrier()

    def erow(r, _):
        for cj in range(4):
            z_v[r, pl.ds(cj * L, L)] = jnp.exp(z_v[r, pl.ds(cj * L, L)] - m)
        return 0
    lax.fori_loop(0, UW, erow, 0)

    def sfire(r, _):
        pltpu.async_copy(z_v.at[r], shared.at[eg_v.at[r]], sem, add=True)
        return 0
    lax.fori_loop(0, UW, sfire, 0)

    def sdrain(r, _):
        pltpu.make_async_copy(
            z_v.at[r], shared.at[eg_v.at[r]], sem).wait()
        return 0
    lax.fori_loop(0, UW, sdrain, 0)

    @pl.when(wid < UC_EXTRA)
    def _extra():
        eu = UW * NW + wid
        pltpu.sync_copy(z_hbm.at[pl.ds(eu, 1)], ex_z)
        pltpu.sync_copy(eg_hbm.at[pl.ds(eu, 1)], ex_eg)
        for j in range(4):
            ex_z[0, pl.ds(j * L, L)] = jnp.exp(ex_z[0, pl.ds(j * L, L)] - m)
        pltpu.sync_copy(ex_z.at[0], shared.at[ex_eg.at[0]], add=True)

    plsc.subcore_barrier()

    @pl.when(lax.axis_index("s") == 0)
    def _out():
        pltpu.sync_copy(shared, sums_hbm.at[lax.axis_index("c")])


@functools.partial(
    pl.kernel,
    out_type=jax.ShapeDtypeStruct((US, 64), jnp.float32),
    mesh=_mesh,
    compiler_params=_params,
    scratch_types=(
        pltpu.VMEM((NW, L), jnp.float32),
        pltpu.VMEM((NC, NSEG), jnp.float32),
        pltpu.VMEM((NSEG,), jnp.float32),
        pltpu.VMEM((384,), jnp.int32),
        pltpu.VMEM((SW, 64), jnp.int32),
        pltpu.VMEM((SW, 64), jnp.float32),
        pltpu.VMEM((SW, 64), jnp.int32),
        pltpu.VMEM((SW, 64), jnp.float32),
        pltpu.VMEM((1, 64), jnp.int32),
        pltpu.VMEM((1, 64), jnp.float32),
        pltpu.VMEM((1, 64), jnp.int32),
        pltpu.VMEM((1, 64), jnp.float32),
        pltpu.SemaphoreType.DMA,
    ),
)
def _k3(samp_hbm, zf_hbm, egf_hbm, pmax_hbm, sums_hbm, out_hbm,
        pm_all, su_v, stab_v, rows_v, ca_v, zc_v, egc_v, o_v,
        ex_ca, ex_z, ex_eg, ex_o, sem):
    wid = _wid()
    s0 = wid * SW
    iota = lax.iota(jnp.int32, L)
    pltpu.sync_copy(pmax_hbm, pm_all)
    m = _global_max(pm_all)
    pltpu.sync_copy(sums_hbm, su_v)
    for t in range(NSEG // L):
        stab_v[pl.ds(t * L, L)] = (
            su_v[0, pl.ds(t * L, L)] + su_v[1, pl.ds(t * L, L)])

    def unit(b, _):
        pltpu.sync_copy(samp_hbm.at[pl.ds((s0 + b) * 384, 384)], rows_v)
        for j in range(4):
            ca_v[b, pl.ds(j * L, L)] = plsc.load_gather(
                rows_v, [(j * L + iota) * 6 + 5])
        return 0
    lax.fori_loop(0, SW, unit, 0)

    def gfire(r, _):
        pltpu.async_copy(zf_hbm.at[ca_v.at[r]], zc_v.at[r], sem)
        pltpu.async_copy(egf_hbm.at[ca_v.at[r]], egc_v.at[r], sem)
        return 0
    lax.fori_loop(0, SW, gfire, 0)

    def gdrain(r, _):
        pltpu.make_async_copy(zf_hbm.at[ca_v.at[r]], zc_v.at[r], sem).wait()
        pltpu.make_async_copy(egf_hbm.at[ca_v.at[r]], egc_v.at[r], sem).wait()
        return 0
    lax.fori_loop(0, SW, gdrain, 0)

    def crow(r, _):
        for cj in range(4):
            sl = pl.ds(cj * L, L)
            s16 = plsc.load_gather(stab_v, [egc_v[r, sl]])
            y = jnp.exp(zc_v[r, sl] - m) / s16
            o_v[r, sl] = (1.0 - y) + y
        return 0
    lax.fori_loop(0, SW, crow, 0)
    pltpu.sync_copy(o_v, out_hbm.at[pl.ds(s0, SW)])

    @pl.when(wid < US_EXTRA)
    def _extra():
        eu = SW * NW + wid
        pltpu.sync_copy(samp_hbm.at[pl.ds(eu * 384, 384)], rows_v)
        for j in range(4):
            ex_ca[0, pl.ds(j * L, L)] = plsc.load_gather(
                rows_v, [(j * L + iota) * 6 + 5])
        pltpu.sync_copy(zf_hbm.at[ex_ca.at[0]], ex_z.at[0])
        pltpu.sync_copy(egf_hbm.at[ex_ca.at[0]], ex_eg.at[0])
        for j in range(4):
            sl = pl.ds(j * L, L)
            s16 = plsc.load_gather(stab_v, [ex_eg[0, sl]])
            y = jnp.exp(ex_z[0, sl] - m) / s16
            ex_o[0, sl] = (1.0 - y) + y
        pltpu.sync_copy(ex_o, out_hbm.at[pl.ds(eu, 1)])


def kernel(candidate_edges, loglog_u, sampled_edges, edges_logits):
    cand_flat = candidate_edges.reshape(-1)
    u2 = loglog_u.reshape(UC, 64)
    samp_flat = sampled_edges.reshape(-1)
    z2, eg2, pmax = _k1(cand_flat, u2, edges_logits)
    sums = _k2(z2, eg2, pmax)
    out2 = _k3(samp_flat, z2.reshape(-1), eg2.reshape(-1), pmax, sums)
    return out2.reshape(-1)


# speedup vs baseline: 18.4062x; 1.0050x over previous
import functools

import jax
import jax.numpy as jnp
from jax import lax
from jax.experimental import pallas as pl
from jax.experimental.pallas import tpu as pltpu
from jax.experimental.pallas import tpu_sc as plsc

NFE = 6400000
NCAND = 1000000
NSAMP = 200000
NSEG = 1024

NC, NS, L = 2, 16, 16
NW = NC * NS

UC = NCAND // 64
UW = UC // NW
CW = UW * 64
UC_EXTRA = UC - UW * NW
US = NSAMP // 64
SW = US // NW
SC_N = SW * 64
US_EXTRA = US - SW * NW

_mesh = plsc.VectorSubcoreMesh(
    core_axis_name="c", subcore_axis_name="s",
    num_cores=NC, num_subcores=NS)

_NEG_BIG = -3.0e38

_params = pltpu.CompilerParams(
    needs_layout_passes=False, use_tc_tiling_on_sc=False)


def _wid():
    return lax.axis_index("s") * NC + lax.axis_index("c")


def _global_max(pm_all):
    def mrow(r, mx):
        return jnp.maximum(mx, pm_all[r, :])
    mx16 = lax.fori_loop(0, NW, mrow, jnp.full((L,), _NEG_BIG, jnp.float32))
    return jnp.max(mx16)


@functools.partial(
    pl.kernel,
    out_type=(
        jax.ShapeDtypeStruct((NCAND,), jnp.float32),
        jax.ShapeDtypeStruct((NCAND,), jnp.int32),
        jax.ShapeDtypeStruct((NW, L), jnp.float32),
    ),
    mesh=_mesh,
    compiler_params=_params,
    scratch_types=(
        pltpu.VMEM((2560,), jnp.int32),
        pltpu.VMEM((CW,), jnp.int32),
        pltpu.VMEM((CW,), jnp.int32),
        pltpu.VMEM((CW,), jnp.float32),
        pltpu.VMEM((CW,), jnp.float32),
        pltpu.VMEM((320,), jnp.int32),
        pltpu.VMEM((64,), jnp.int32),
        pltpu.VMEM((64,), jnp.int32),
        pltpu.VMEM((64,), jnp.float32),
        pltpu.VMEM((64,), jnp.float32),
        pltpu.VMEM((L,), jnp.float32),
        pltpu.VMEM((L,), jnp.float32),
        pltpu.SemaphoreType.DMA,
    ),
)
def _k1(cand_hbm, u_hbm, logits_hbm, z_hbm, eg_hbm, pmax_hbm,
        rows_v, ei_v, eg_v, lg_v, u_v,
        ex_rows, ex_ei, ex_eg, ex_lg, ex_u, pm_v, max_ref, sem):
    wid = _wid()
    c0 = wid * CW
    iota = lax.iota(jnp.int32, L)
    max_ref[:] = jnp.full((L,), _NEG_BIG, jnp.float32)

    def batch_body(b, _):
        pltpu.sync_copy(cand_hbm.at[pl.ds(c0 * 5 + b * 2560, 2560)], rows_v)
        for un in range(8):
            for j in range(4):
                base16 = (un * 64 + j * L + iota) * 5
                g_eg = plsc.load_gather(rows_v, [base16])
                g_ei = plsc.load_gather(rows_v, [base16 + 1])
                p = b * 512 + un * 64 + j * L
                eg_v[pl.ds(p, L)] = g_eg
                ei_v[pl.ds(p, L)] = g_ei
        return 0
    lax.fori_loop(0, UW // 8, batch_body, 0)

    gather = pltpu.async_copy(logits_hbm.at[ei_v], lg_v, sem)
    pltpu.sync_copy(u_hbm.at[pl.ds(c0, CW)], u_v)

    @pl.when(wid < UC_EXTRA)
    def _extra():
        e0 = (UW * NW + wid) * 64
        pltpu.sync_copy(cand_hbm.at[pl.ds(e0 * 5, 320)], ex_rows)
        for j in range(4):
            base16 = (j * L + iota) * 5
            ex_eg[pl.ds(j * L, L)] = plsc.load_gather(ex_rows, [base16])
            ex_ei[pl.ds(j * L, L)] = plsc.load_gather(ex_rows, [base16 + 1])
        pltpu.sync_copy(u_hbm.at[pl.ds(e0, 64)], ex_u)
        pltpu.sync_copy(logits_hbm.at[ex_ei], ex_lg)
        for j in range(4):
            zj = ex_lg[pl.ds(j * L, L)] + ex_u[pl.ds(j * L, L)]
            ex_lg[pl.ds(j * L, L)] = zj
            max_ref[:] = jnp.maximum(max_ref[:], zj)
        pltpu.sync_copy(ex_lg, z_hbm.at[pl.ds(e0, 64)])
        pltpu.sync_copy(ex_eg, eg_hbm.at[pl.ds(e0, 64)])

    gather.wait()

    def zrow(r, mx):
        for cj in range(4):
            sl = pl.ds(r * 64 + cj * L, L)
            zj = lg_v[sl] + u_v[sl]
            lg_v[sl] = zj
            mx = jnp.maximum(mx, zj)
        return mx
    mx = lax.fori_loop(0, UW, zrow, max_ref[:])

    pm_v[:] = mx
    pltpu.sync_copy(pm_v, pmax_hbm.at[wid])
    pltpu.sync_copy(lg_v, z_hbm.at[pl.ds(c0, CW)])
    pltpu.sync_copy(eg_v, eg_hbm.at[pl.ds(c0, CW)])


@functools.partial(
    pl.kernel,
    out_type=jax.ShapeDtypeStruct((NC, NSEG), jnp.float32),
    mesh=_mesh,
    compiler_params=_params,
    scratch_types=(
        pltpu.VMEM((NW, L), jnp.float32),
        pltpu.VMEM((CW,), jnp.float32),
        pltpu.VMEM((CW,), jnp.int32),
        pltpu.VMEM((64,), jnp.float32),
        pltpu.VMEM((64,), jnp.int32),
        pltpu.VMEM((NSEG,), jnp.float32),
        pltpu.VMEM_SHARED((NSEG,), jnp.float32),
        pltpu.SemaphoreType.DMA,
    ),
)
def _k2(z_hbm, eg_hbm, pmax_hbm, sums_hbm,
        pm_all, z_v, eg_v, ex_z, ex_eg, zeros_v, shared, sem):
    wid = _wid()
    c0 = wid * CW
    pltpu.sync_copy(pmax_hbm, pm_all)
    m = _global_max(pm_all)

    pltpu.sync_copy(z_hbm.at[pl.ds(c0, CW)], z_v)
    pltpu.sync_copy(eg_hbm.at[pl.ds(c0, CW)], eg_v)

    for t in range(NSEG // L):
        zeros_v[pl.ds(t * L, L)] = jnp.zeros((L,), jnp.float32)

    @pl.when(lax.axis_index("s") == 0)
    def _init():
        pltpu.sync_copy(zeros_v, shared)
    plsc.subcore_barrier()

    def erow(r, _):
        for cj in range(4):
            sl = pl.ds(r * 64 + cj * L, L)
            z_v[sl] = jnp.exp(z_v[sl] - m)
        return 0
    lax.fori_loop(0, UW, erow, 0)

    pltpu.async_copy(z_v, shared.at[eg_v], sem, add=True).wait()

    @pl.when(wid < UC_EXTRA)
    def _extra():
        e0 = (UW * NW + wid) * 64
        pltpu.sync_copy(z_hbm.at[pl.ds(e0, 64)], ex_z)
        pltpu.sync_copy(eg_hbm.at[pl.ds(e0, 64)], ex_eg)
        for j in range(4):
            ex_z[pl.ds(j * L, L)] = jnp.exp(ex_z[pl.ds(j * L, L)] - m)
        pltpu.sync_copy(ex_z, shared.at[ex_eg], add=True)

    plsc.subcore_barrier()

    @pl.when(lax.axis_index("s") == 0)
    def _out():
        pltpu.sync_copy(shared, sums_hbm.at[lax.axis_index("c")])


@functools.partial(
    pl.kernel,
    out_type=jax.ShapeDtypeStruct((NSAMP,), jnp.float32),
    mesh=_mesh,
    compiler_params=_params,
    scratch_types=(
        pltpu.VMEM((NW, L), jnp.float32),
        pltpu.VMEM((NC, NSEG), jnp.float32),
        pltpu.VMEM((NSEG,), jnp.float32),
        pltpu.VMEM((384,), jnp.int32),
        pltpu.VMEM((SC_N,), jnp.int32),
        pltpu.VMEM((SC_N,), jnp.float32),
        pltpu.VMEM((SC_N,), jnp.int32),
        pltpu.VMEM((SC_N,), jnp.float32),
        pltpu.VMEM((64,), jnp.int32),
        pltpu.VMEM((64,), jnp.float32),
        pltpu.VMEM((64,), jnp.int32),
        pltpu.VMEM((64,), jnp.float32),
        pltpu.SemaphoreType.DMA,
    ),
)
def _k3(samp_hbm, zf_hbm, egf_hbm, pmax_hbm, sums_hbm, out_hbm,
        pm_all, su_v, stab_v, rows_v, ca_v, zc_v, egc_v, o_v,
        ex_ca, ex_z, ex_eg, ex_o, sem):
    wid = _wid()
    s0 = wid * SC_N
    iota = lax.iota(jnp.int32, L)
    pltpu.sync_copy(pmax_hbm, pm_all)
    m = _global_max(pm_all)
    pltpu.sync_copy(sums_hbm, su_v)
    for t in range(NSEG // L):
        stab_v[pl.ds(t * L, L)] = (
            su_v[0, pl.ds(t * L, L)] + su_v[1, pl.ds(t * L, L)])

    def unit(b, _):
        pltpu.sync_copy(samp_hbm.at[pl.ds(s0 * 6 + b * 384, 384)], rows_v)
        for j in range(4):
            ca_v[pl.ds(b * 64 + j * L, L)] = plsc.load_gather(
                rows_v, [(j * L + iota) * 6 + 5])
        return 0
    lax.fori_loop(0, SW, unit, 0)

    g1 = pltpu.async_copy(zf_hbm.at[ca_v], zc_v, sem)
    g1.wait()
    g2 = pltpu.async_copy(egf_hbm.at[ca_v], egc_v, sem)
    g2.wait()

    def crow(r, _):
        for cj in range(4):
            sl = pl.ds(r * 64 + cj * L, L)
            s16 = plsc.load_gather(stab_v, [egc_v[sl]])
            y = jnp.exp(zc_v[sl] - m) / s16
            o_v[sl] = (1.0 - y) + y
        return 0
    lax.fori_loop(0, SW, crow, 0)
    pltpu.sync_copy(o_v, out_hbm.at[pl.ds(s0, SC_N)])

    @pl.when(wid < US_EXTRA)
    def _extra():
        e0 = (SW * NW + wid) * 64
        pltpu.sync_copy(samp_hbm.at[pl.ds(e0 * 6, 384)], rows_v)
        for j in range(4):
            ex_ca[pl.ds(j * L, L)] = plsc.load_gather(
                rows_v, [(j * L + iota) * 6 + 5])
        pltpu.sync_copy(zf_hbm.at[ex_ca], ex_z)
        pltpu.sync_copy(egf_hbm.at[ex_ca], ex_eg)
        for j in range(4):
            sl = pl.ds(j * L, L)
            s16 = plsc.load_gather(stab_v, [ex_eg[sl]])
            y = jnp.exp(ex_z[sl] - m) / s16
            ex_o[sl] = (1.0 - y) + y
        pltpu.sync_copy(ex_o, out_hbm.at[pl.ds(e0, 64)])


def kernel(candidate_edges, loglog_u, sampled_edges, edges_logits):
    cand_flat = candidate_edges.reshape(-1)
    samp_flat = sampled_edges.reshape(-1)
    z, eg, pmax = _k1(cand_flat, loglog_u, edges_logits)
    sums = _k2(z, eg, pmax)
    return _k3(samp_flat, z, eg, pmax, sums)
